# baseline (speedup 1.0000x reference)
import functools

import jax
import jax.numpy as jnp
from jax import lax
from jax.experimental import pallas as pl
from jax.experimental.pallas import tpu as pltpu
from jax.experimental.pallas import tpu_sc as plsc

WIN_AREA = 1024
NPOS = WIN_AREA * WIN_AREA
NHEADS = 16
NDIST = 3969

NC, NS, L = 2, 16, 16
NW = NC * NS
POS_PER_W = NPOS // NW
CHUNK = 1024
N_CHUNKS = POS_PER_W // CHUNK
UNROLL = 4


def _sc_bias_kernel(table_hbm, idx_hbm, bias_hbm, table_v, idx_v,
                    buf_a, buf_b, sem_t, sem_i, sem_a, sem_b):
    wid = lax.axis_index("s") * NC + lax.axis_index("c")
    base = wid * POS_PER_W
    row0 = wid * N_CHUNKS

    tcopy = pltpu.async_copy(table_hbm, table_v, sem_t)
    icopy = pltpu.async_copy(idx_hbm.at[pl.ds(base, POS_PER_W)], idx_v, sem_i)
    tcopy.wait()
    icopy.wait()

    def gather_chunk(c, buf):
        @plsc.parallel_loop(0, CHUNK // L, unroll=UNROLL)
        def group_body(k):
            iv = idx_v[pl.ds(c * CHUNK + k * L, L)] * NHEADS
            for h in range(NHEADS):
                buf[h, pl.ds(k * L, L)] = plsc.load_gather(table_v, [iv + h])

    def put_chunk(c, buf, sem):
        pltpu.async_copy(buf, bias_hbm.at[:, row0 + c], sem)

    def wait_chunk(buf, sem):
        pltpu.make_async_copy(buf, bias_hbm.at[:, row0], sem).wait()

    gather_chunk(0, buf_a)
    put_chunk(0, buf_a, sem_a)
    gather_chunk(1, buf_b)
    put_chunk(1, buf_b, sem_b)

    def pair_body(p, _):
        c = p * 2
        wait_chunk(buf_a, sem_a)
        gather_chunk(c, buf_a)
        put_chunk(c, buf_a, sem_a)
        wait_chunk(buf_b, sem_b)
        gather_chunk(c + 1, buf_b)
        put_chunk(c + 1, buf_b, sem_b)
        return ()

    lax.fori_loop(1, N_CHUNKS // 2, pair_body, (), unroll=False)
    wait_chunk(buf_a, sem_a)
    wait_chunk(buf_b, sem_b)


def _sc_build_bias(table, idx):
    mesh = plsc.VectorSubcoreMesh(core_axis_name="c", subcore_axis_name="s")
    return pl.kernel(
        _sc_bias_kernel,
        out_type=jax.ShapeDtypeStruct((NHEADS, WIN_AREA, WIN_AREA), jnp.float32),
        mesh=mesh,
        compiler_params=pltpu.CompilerParams(needs_layout_passes=False),
        scratch_types=[
            pltpu.VMEM((NDIST * NHEADS,), jnp.float32),
            pltpu.VMEM((POS_PER_W,), jnp.int32),
            pltpu.VMEM((NHEADS, CHUNK), jnp.float32),
            pltpu.VMEM((NHEADS, CHUNK), jnp.float32),
            pltpu.SemaphoreType.DMA,
            pltpu.SemaphoreType.DMA,
            pltpu.SemaphoreType.DMA,
            pltpu.SemaphoreType.DMA,
        ],
    )(table, idx)


BI = 8


def _tc_add_kernel(attn_ref, bias_ref, out_ref):
    out_ref[...] = attn_ref[...] + bias_ref[...][None]


def _tc_add(attn, bias3):
    nb = attn.shape[0]
    return pl.pallas_call(
        _tc_add_kernel,
        grid=(WIN_AREA // BI,),
        in_specs=[
            pl.BlockSpec((nb, NHEADS, BI, WIN_AREA), lambda ib: (0, 0, ib, 0)),
            pl.BlockSpec((NHEADS, BI, WIN_AREA), lambda ib: (0, ib, 0)),
        ],
        out_specs=pl.BlockSpec((nb, NHEADS, BI, WIN_AREA), lambda ib: (0, 0, ib, 0)),
        out_shape=jax.ShapeDtypeStruct(attn.shape, attn.dtype),
    )(attn, bias3)


@jax.jit
def kernel(attn, relative_position_bias_table, relative_position_index):
    bias3 = _sc_build_bias(relative_position_bias_table.reshape(-1),
                           relative_position_index)
    return _tc_add(attn, bias3)


# speedup vs baseline: 1.0834x; 1.0834x over previous
import jax
import jax.numpy as jnp
from jax import lax
from jax.experimental import pallas as pl
from jax.experimental.pallas import tpu as pltpu
from jax.experimental.pallas import tpu_sc as plsc

WIN_AREA = 1024
NHEADS = 16
NDIST = 3969

NC, NS, L = 2, 16, 16
NW = NC * NS
CHUNK = WIN_AREA
UNROLL = 4

NPIECES = 4
ROWS_P = WIN_AREA // NPIECES
N_CH_P = ROWS_P // NW
POS_P = N_CH_P * CHUNK


def _sc_bias_kernel(k, table_hbm, idx_hbm, bias_hbm, table_v, idx_v,
                    buf_a, buf_b, sem_t, sem_i, sem_a, sem_b):
    wid = lax.axis_index("s") * NC + lax.axis_index("c")
    row0 = wid * N_CH_P
    base = (k * ROWS_P + row0) * WIN_AREA

    tcopy = pltpu.async_copy(table_hbm, table_v, sem_t)
    icopy = pltpu.async_copy(idx_hbm.at[pl.ds(base, POS_P)], idx_v, sem_i)
    tcopy.wait()
    icopy.wait()

    def gather_chunk(c, buf):
        @plsc.parallel_loop(0, CHUNK // L, unroll=UNROLL)
        def group_body(g):
            iv = idx_v[pl.ds(c * CHUNK + g * L, L)] * NHEADS
            for h in range(NHEADS):
                buf[h, pl.ds(g * L, L)] = plsc.load_gather(table_v, [iv + h])

    def put_chunk(c, buf, sem):
        pltpu.async_copy(buf, bias_hbm.at[:, row0 + c], sem)

    def wait_chunk(buf, sem):
        pltpu.make_async_copy(buf, bias_hbm.at[:, row0], sem).wait()

    gather_chunk(0, buf_a)
    put_chunk(0, buf_a, sem_a)
    gather_chunk(1, buf_b)
    put_chunk(1, buf_b, sem_b)

    def pair_body(p, _):
        c = p * 2
        wait_chunk(buf_a, sem_a)
        gather_chunk(c, buf_a)
        put_chunk(c, buf_a, sem_a)
        wait_chunk(buf_b, sem_b)
        gather_chunk(c + 1, buf_b)
        put_chunk(c + 1, buf_b, sem_b)
        return ()

    lax.fori_loop(1, N_CH_P // 2, pair_body, (), unroll=False)
    wait_chunk(buf_a, sem_a)
    wait_chunk(buf_b, sem_b)


def _sc_build_bias_piece(k, table, idx):
    mesh = plsc.VectorSubcoreMesh(core_axis_name="c", subcore_axis_name="s")
    return pl.kernel(
        lambda *refs: _sc_bias_kernel(k, *refs),
        out_type=jax.ShapeDtypeStruct((NHEADS, ROWS_P, WIN_AREA),
                                      jnp.float32),
        mesh=mesh,
        compiler_params=pltpu.CompilerParams(needs_layout_passes=False),
        scratch_types=[
            pltpu.VMEM((NDIST * NHEADS,), jnp.float32),
            pltpu.VMEM((POS_P,), jnp.int32),
            pltpu.VMEM((NHEADS, CHUNK), jnp.float32),
            pltpu.VMEM((NHEADS, CHUNK), jnp.float32),
            pltpu.SemaphoreType.DMA,
            pltpu.SemaphoreType.DMA,
            pltpu.SemaphoreType.DMA,
            pltpu.SemaphoreType.DMA,
        ],
        name=f"sc_bias_gather_{k}",
    )(table, idx)


BI = 16


def _tc_add_first_kernel(attn_ref, bias_ref, out_ref):
    out_ref[...] = attn_ref[...] + bias_ref[...][None]


def _tc_add_chain_kernel(prev_ref, attn_ref, bias_ref, out_ref):
    del prev_ref
    out_ref[...] = attn_ref[...] + bias_ref[...][None]


def _tc_add_piece(k, prev_out, attn, bias_p):
    nb = attn.shape[0]
    nblk = ROWS_P // BI
    data_spec = pl.BlockSpec((nb, NHEADS, BI, WIN_AREA),
                             lambda ib: (0, 0, k * nblk + ib, 0))
    bias_spec = pl.BlockSpec((NHEADS, BI, WIN_AREA), lambda ib: (0, ib, 0))
    out_shape = jax.ShapeDtypeStruct(attn.shape, attn.dtype)
    if k == 0:
        return pl.pallas_call(
            _tc_add_first_kernel,
            grid=(nblk,),
            in_specs=[data_spec, bias_spec],
            out_specs=data_spec,
            out_shape=out_shape,
        )(attn, bias_p)
    prev_spec = pl.BlockSpec((1, 1, 8, 128), lambda ib: (0, 0, 0, 0))
    return pl.pallas_call(
        _tc_add_chain_kernel,
        grid=(nblk,),
        in_specs=[prev_spec, data_spec, bias_spec],
        out_specs=data_spec,
        out_shape=out_shape,
        input_output_aliases={0: 0},
    )(prev_out, attn, bias_p)


@jax.jit
def kernel(attn, relative_position_bias_table, relative_position_index):
    table_flat = relative_position_bias_table.reshape(-1)
    biases = [_sc_build_bias_piece(k, table_flat, relative_position_index)
              for k in range(NPIECES)]
    out = None
    for k in range(NPIECES):
        out = _tc_add_piece(k, out, attn, biases[k])
    return out
